# Initial kernel scaffold; baseline (speedup 1.0000x reference)
import jax
import jax.numpy as jnp
from jax.experimental import pallas as pl


def kernel(pos_embed_ids, lp_embeds, token_type_ids):
    raise NotImplementedError("write your pallas kernel here")


# speedup vs baseline: 20.6118x; 20.6118x over previous
import functools

import jax
import jax.numpy as jnp
from jax import lax
from jax.experimental import pallas as pl
from jax.experimental.pallas import tpu as pltpu
from jax.experimental.pallas import tpu_sc as plsc

ATOM = 1
BOND = 2

B, L, K, D = 128, 512, 4, 64
N = B * L
NC, NS = 2, 16
NW = NC * NS
TOK_W = N // NW
ROWS_W = TOK_W * K
STEP_ROWS = 128
STEP_TOK = STEP_ROWS // K
STEPS = ROWS_W // STEP_ROWS
LANES = 16


def _body(ids_hbm, tt_hbm, table_hbm, out_hbm,
          gidx, ttv, maskf, bin0, bin1, bout0, bout1,
          gs0, gs1, ws0, ws1):
    wid = lax.axis_index("s") * NC + lax.axis_index("c")
    tok0 = wid * TOK_W
    row0 = wid * ROWS_W

    pltpu.sync_copy(ids_hbm.at[pl.ds(row0, ROWS_W)], gidx)
    pltpu.sync_copy(tt_hbm.at[pl.ds(tok0, TOK_W)], ttv)

    def idx_body(j, _):
        off = (tok0 + j * (LANES // K)) // L * L
        sl = pl.ds(j * LANES, LANES)
        gidx[sl] = gidx[sl] + off
        return _
    lax.fori_loop(0, ROWS_W // LANES, idx_body, 0, unroll=4)

    def mask_body(j, _):
        sl = pl.ds(j * LANES, LANES)
        v = ttv[sl]
        m = (v == ATOM) | (v == BOND)
        maskf[sl] = jnp.where(m, 1.0, 0.0).astype(jnp.float32)
        return _
    lax.fori_loop(0, TOK_W // LANES, mask_body, 0, unroll=4)

    def fire_gather(step, buf, sem):
        pltpu.make_async_copy(
            table_hbm.at[gidx.at[pl.ds(step * STEP_ROWS, STEP_ROWS)]],
            buf, sem).start()

    def wait_gather(buf, sem):
        pltpu.make_async_copy(
            table_hbm.at[gidx.at[pl.ds(0, STEP_ROWS)]], buf, sem).wait()

    def fire_out(step, buf, sem):
        pltpu.make_async_copy(
            buf, out_hbm.at[pl.ds(row0 + step * STEP_ROWS, STEP_ROWS)],
            sem).start()

    def wait_out(buf, sem):
        pltpu.make_async_copy(
            buf, out_hbm.at[pl.ds(row0, STEP_ROWS)], sem).wait()

    def mask_mul(step, src, dst):
        def grp_body(g, _):
            mvec = maskf[pl.ds((step * STEP_TOK + g * LANES), LANES)]
            for t in range(LANES):
                iv = jnp.full((LANES,), t, jnp.int32)
                splat = mvec.at[iv].get(mode="promise_in_bounds")
                for q in range(K):
                    r = (g * LANES + t) * K + q
                    for c in range(D // LANES):
                        sl = pl.ds(c * LANES, LANES)
                        dst[r, sl] = src[r, sl] * splat
            return _
        lax.fori_loop(0, STEP_TOK // LANES, grp_body, 0)

    fire_gather(0, bin0, gs0)
    fire_gather(1, bin1, gs1)

    def loop_body(i, _):
        a = 2 * i

        wait_gather(bin0, gs0)

        @pl.when(i > 0)
        def _w0():
            wait_out(bout0, ws0)
        mask_mul(a, bin0, bout0)

        @pl.when(i < STEPS // 2 - 1)
        def _g0():
            fire_gather(a + 2, bin0, gs0)
        fire_out(a, bout0, ws0)

        wait_gather(bin1, gs1)

        @pl.when(i > 0)
        def _w1():
            wait_out(bout1, ws1)
        mask_mul(a + 1, bin1, bout1)

        @pl.when(i < STEPS // 2 - 1)
        def _g1():
            fire_gather(a + 3, bin1, gs1)
        fire_out(a + 1, bout1, ws1)
        return _

    lax.fori_loop(0, STEPS // 2, loop_body, 0)
    wait_out(bout0, ws0)
    wait_out(bout1, ws1)


@jax.jit
def _run(ids_flat, tt_flat, table):
    mesh = plsc.VectorSubcoreMesh(
        core_axis_name="c", subcore_axis_name="s",
        num_cores=NC, num_subcores=NS)
    return pl.kernel(
        _body,
        out_type=jax.ShapeDtypeStruct((N * K, D), jnp.float32),
        mesh=mesh,
        compiler_params=pltpu.CompilerParams(use_tc_tiling_on_sc=False),
        scratch_types=[
            pltpu.VMEM((ROWS_W,), jnp.int32),
            pltpu.VMEM((TOK_W,), jnp.int32),
            pltpu.VMEM((TOK_W,), jnp.float32),
            pltpu.VMEM((STEP_ROWS, D), jnp.float32),
            pltpu.VMEM((STEP_ROWS, D), jnp.float32),
            pltpu.VMEM((STEP_ROWS, D), jnp.float32),
            pltpu.VMEM((STEP_ROWS, D), jnp.float32),
            pltpu.SemaphoreType.DMA,
            pltpu.SemaphoreType.DMA,
            pltpu.SemaphoreType.DMA,
            pltpu.SemaphoreType.DMA,
        ],
    )(ids_flat, tt_flat, table)


def kernel(pos_embed_ids, lp_embeds, token_type_ids):
    ids_flat = pos_embed_ids.astype(jnp.int32).reshape(N * K)
    tt_flat = token_type_ids.astype(jnp.int32).reshape(N)
    table = lp_embeds.reshape(N, D)
    out = _run(ids_flat, tt_flat, table)
    return out.reshape(B, L, K * D)


# speedup vs baseline: 20.6717x; 1.0029x over previous
import functools

import jax
import jax.numpy as jnp
from jax import lax
from jax.experimental import pallas as pl
from jax.experimental.pallas import tpu as pltpu
from jax.experimental.pallas import tpu_sc as plsc

ATOM = 1
BOND = 2

B, L, K, D = 128, 512, 4, 64
N = B * L
NC, NS = 2, 16
NW = NC * NS
TOK_W = N // NW
ROWS_W = TOK_W * K
STEP_ROWS = 128
STEP_TOK = STEP_ROWS // K
STEPS = ROWS_W // STEP_ROWS
LANES = 16


def _body(ids_hbm, tt_hbm, table_hbm, out_hbm,
          gidx, ttv, maskf, bin0, bin1, bout0, bout1,
          gs0, gs1, ws0, ws1):
    wid = lax.axis_index("s") * NC + lax.axis_index("c")
    tok0 = wid * TOK_W
    row0 = wid * ROWS_W

    pltpu.sync_copy(ids_hbm.at[pl.ds(row0, ROWS_W)], gidx)
    pltpu.sync_copy(tt_hbm.at[pl.ds(tok0, TOK_W)], ttv)

    def idx_body(j, _):
        off = (tok0 + j * (LANES // K)) // L * L
        sl = pl.ds(j * LANES, LANES)
        gidx[sl] = gidx[sl] + off
        return _
    lax.fori_loop(0, ROWS_W // LANES, idx_body, 0, unroll=4)

    def mask_body(j, _):
        sl = pl.ds(j * LANES, LANES)
        v = ttv[sl]
        m = (v == ATOM) | (v == BOND)
        maskf[sl] = jnp.where(m, 1.0, 0.0).astype(jnp.float32)
        return _
    lax.fori_loop(0, TOK_W // LANES, mask_body, 0, unroll=4)

    def fire_gather(step, buf, sem):
        pltpu.make_async_copy(
            table_hbm.at[gidx.at[pl.ds(step * STEP_ROWS, STEP_ROWS)]],
            buf, sem).start()

    def wait_gather(buf, sem):
        pltpu.make_async_copy(
            table_hbm.at[gidx.at[pl.ds(0, STEP_ROWS)]], buf, sem).wait()

    def fire_out(step, buf, sem):
        pltpu.make_async_copy(
            buf, out_hbm.at[pl.ds(tok0 + step * STEP_TOK, STEP_TOK)],
            sem).start()

    def wait_out(buf, sem):
        pltpu.make_async_copy(
            buf, out_hbm.at[pl.ds(0, STEP_TOK)], sem).wait()

    def mask_mul(step, src, dst):
        def grp_body(g, _):
            mvec = maskf[pl.ds((step * STEP_TOK + g * LANES), LANES)]
            for t in range(LANES):
                iv = jnp.full((LANES,), t, jnp.int32)
                splat = mvec.at[iv].get(mode="promise_in_bounds")
                tok = g * LANES + t
                for q in range(K):
                    for c in range(D // LANES):
                        dst[tok, pl.ds(q * D + c * LANES, LANES)] = (
                            src[tok * K + q, pl.ds(c * LANES, LANES)] * splat)
            return _
        lax.fori_loop(0, STEP_TOK // LANES, grp_body, 0)

    fire_gather(0, bin0, gs0)
    fire_gather(1, bin1, gs1)

    def loop_body(i, _):
        a = 2 * i

        wait_gather(bin0, gs0)

        @pl.when(i > 0)
        def _w0():
            wait_out(bout0, ws0)
        mask_mul(a, bin0, bout0)

        @pl.when(i < STEPS // 2 - 1)
        def _g0():
            fire_gather(a + 2, bin0, gs0)
        fire_out(a, bout0, ws0)

        wait_gather(bin1, gs1)

        @pl.when(i > 0)
        def _w1():
            wait_out(bout1, ws1)
        mask_mul(a + 1, bin1, bout1)

        @pl.when(i < STEPS // 2 - 1)
        def _g1():
            fire_gather(a + 3, bin1, gs1)
        fire_out(a + 1, bout1, ws1)
        return _

    lax.fori_loop(0, STEPS // 2, loop_body, 0)
    wait_out(bout0, ws0)
    wait_out(bout1, ws1)


@jax.jit
def _run(ids_flat, tt_flat, table):
    mesh = plsc.VectorSubcoreMesh(
        core_axis_name="c", subcore_axis_name="s",
        num_cores=NC, num_subcores=NS)
    return pl.kernel(
        _body,
        out_type=jax.ShapeDtypeStruct((N, K * D), jnp.float32),
        mesh=mesh,
        compiler_params=pltpu.CompilerParams(use_tc_tiling_on_sc=False),
        scratch_types=[
            pltpu.VMEM((ROWS_W,), jnp.int32),
            pltpu.VMEM((TOK_W,), jnp.int32),
            pltpu.VMEM((TOK_W,), jnp.float32),
            pltpu.VMEM((STEP_ROWS, D), jnp.float32),
            pltpu.VMEM((STEP_ROWS, D), jnp.float32),
            pltpu.VMEM((STEP_TOK, K * D), jnp.float32),
            pltpu.VMEM((STEP_TOK, K * D), jnp.float32),
            pltpu.SemaphoreType.DMA,
            pltpu.SemaphoreType.DMA,
            pltpu.SemaphoreType.DMA,
            pltpu.SemaphoreType.DMA,
        ],
    )(ids_flat, tt_flat, table)


def kernel(pos_embed_ids, lp_embeds, token_type_ids):
    ids_flat = pos_embed_ids.astype(jnp.int32).reshape(N * K)
    tt_flat = token_type_ids.astype(jnp.int32).reshape(N)
    table = lp_embeds.reshape(N, D)
    out = _run(ids_flat, tt_flat, table)
    return out.reshape(B, L, K * D)
